# baseline (speedup 1.0000x reference)
import functools

import jax
import jax.numpy as jnp
from jax import lax
from jax.experimental import pallas as pl
from jax.experimental.pallas import tpu as pltpu
from jax.experimental.pallas import tpu_sc as plsc

N = 4096
A = 768
D = 16384
TOP_K = 32
K_TOT = TOP_K * N
L1_COEFF = 0.0003

NC = 2
NS = 16
L = 16
NW = NC * NS

NBINS = 2048
HIST_W = L * NBINS
SHIFTS = (20, 10, 0)
SPANS = (2048, 1024, 1024)

ELEMS = N * D
ROWS_PER_W = N // NW
WIN_C = 2048
CGRP = D // WIN_C
N_WIN = (ROWS_PER_W // 8) * CGRP


def _prep_body(x_ref, bdec_ref, xn_ref, xe_ref, mean_ref, std_ref):
    x = x_ref[...]
    mu = jnp.mean(x, axis=1, keepdims=True)
    xc = x - mu
    var = jnp.sum(xc * xc, axis=1, keepdims=True) / (A - 1)
    sd = jnp.sqrt(var)
    xn = xc / (sd + 1e-5)
    xn_ref[...] = xn
    xe_ref[...] = xn - bdec_ref[...]
    mean_ref[...] = mu
    std_ref[...] = sd


def _prep(x, b_dec, interpret=False):
    blk = 512
    return pl.pallas_call(
        _prep_body,
        grid=(N // blk,),
        in_specs=[
            pl.BlockSpec((blk, A), lambda i: (i, 0)),
            pl.BlockSpec((1, A), lambda i: (0, 0)),
        ],
        out_specs=[
            pl.BlockSpec((blk, A), lambda i: (i, 0)),
            pl.BlockSpec((blk, A), lambda i: (i, 0)),
            pl.BlockSpec((blk, 1), lambda i: (i, 0)),
            pl.BlockSpec((blk, 1), lambda i: (i, 0)),
        ],
        out_shape=[
            jax.ShapeDtypeStruct((N, A), jnp.float32),
            jax.ShapeDtypeStruct((N, A), jnp.float32),
            jax.ShapeDtypeStruct((N, 1), jnp.float32),
            jax.ShapeDtypeStruct((N, 1), jnp.float32),
        ],
        interpret=interpret,
    )(x, b_dec.reshape(1, A))


def _enc_body(xe_ref, w_ref, acts_ref, s1_ref, s0_ref):
    j = pl.program_id(0)
    a = jnp.dot(xe_ref[...], w_ref[...], preferred_element_type=jnp.float32)
    a = jnp.maximum(a, 0.0)
    acts_ref[...] = a

    @pl.when(j == 0)
    def _():
        s1_ref[0, 0] = 0.0
        s0_ref[0, 0] = 0.0

    s1_ref[0, 0] += jnp.sum(a)
    s0_ref[0, 0] += jnp.sum((a > 0.0).astype(jnp.float32))


def _encode(xe, w_enc, interpret=False):
    blk = 512
    return pl.pallas_call(
        _enc_body,
        grid=(D // blk,),
        in_specs=[
            pl.BlockSpec((N, A), lambda j: (0, 0)),
            pl.BlockSpec((A, blk), lambda j: (0, j)),
        ],
        out_specs=[
            pl.BlockSpec((N, blk), lambda j: (0, j)),
            pl.BlockSpec((1, 1), lambda j: (0, 0), memory_space=pltpu.SMEM),
            pl.BlockSpec((1, 1), lambda j: (0, 0), memory_space=pltpu.SMEM),
        ],
        out_shape=[
            jax.ShapeDtypeStruct((N, D), jnp.float32),
            jax.ShapeDtypeStruct((1, 1), jnp.float32),
            jax.ShapeDtypeStruct((1, 1), jnp.float32),
        ],
        interpret=interpret,
    )(xe, w_enc)


def _hist_body(shift, span, acts_hbm, lo_hbm, out_hbm,
               hist, buf0, buf1, lo_v, sem0, sem1):
    wid = lax.axis_index("s") * NC + lax.axis_index("c")
    row0 = wid * ROWS_PER_W

    def zbody(i, _):
        hist[pl.ds(i * L, L)] = jnp.zeros((L,), jnp.int32)
        return 0
    lax.fori_loop(0, HIST_W // L, zbody, 0, unroll=4)

    pltpu.sync_copy(lo_hbm, lo_v)
    lo = lo_v[...]
    lane_base = lax.iota(jnp.int32, L) * NBINS
    ones = jnp.ones((L,), jnp.int32)

    def win_slice(g):
        rg = g // CGRP
        cg = lax.rem(g, CGRP)
        return acts_hbm.at[pl.ds(row0 + rg * 8, 8), pl.ds(cg * WIN_C, WIN_C)]

    def compute(buf):
        @plsc.parallel_loop(0, WIN_C, step=L * 8)
        def _(c0):
            for r in range(8):
                for u in range(8):
                    c = c0 + u * L
                    vals = buf[r, pl.ds(c, L)]
                    bits = plsc.bitcast(vals, jnp.int32)
                    if shift == SHIFTS[0]:
                        m = bits != 0
                        plsc.addupdate_scatter(
                            hist, [(bits >> shift) + lane_base], ones, mask=m)
                    else:
                        bin_ = (bits >> shift) - lo
                        m = bin_.astype(jnp.uint32) < jnp.uint32(span)
                        plsc.addupdate_scatter(
                            hist, [bin_ + lane_base], ones, mask=m)

    pltpu.make_async_copy(win_slice(0), buf0, sem0).start()
    pltpu.make_async_copy(win_slice(1), buf1, sem1).start()

    def gbody(gg, _):
        g0 = 2 * gg
        pltpu.make_async_copy(win_slice(g0), buf0, sem0).wait()
        compute(buf0)
        pltpu.make_async_copy(win_slice(g0 + 2), buf0, sem0).start()
        pltpu.make_async_copy(win_slice(g0 + 1), buf1, sem1).wait()
        compute(buf1)
        pltpu.make_async_copy(win_slice(g0 + 3), buf1, sem1).start()
        return 0

    lax.fori_loop(0, N_WIN // 2 - 1, gbody, 0)
    pltpu.make_async_copy(win_slice(N_WIN - 2), buf0, sem0).wait()
    compute(buf0)
    pltpu.make_async_copy(win_slice(N_WIN - 1), buf1, sem1).wait()
    compute(buf1)
    pltpu.sync_copy(hist, out_hbm.at[wid])


def _sc_hist(shift, span):
    mesh = plsc.VectorSubcoreMesh(core_axis_name="c", subcore_axis_name="s")
    return pl.kernel(
        functools.partial(_hist_body, shift, span),
        out_type=jax.ShapeDtypeStruct((NW, HIST_W), jnp.int32),
        mesh=mesh,
        compiler_params=pltpu.CompilerParams(needs_layout_passes=False,
                                             use_tc_tiling_on_sc=True),
        scratch_types=[
            pltpu.VMEM((HIST_W,), jnp.int32),
            pltpu.VMEM((8, WIN_C), jnp.float32),
            pltpu.VMEM((8, WIN_C), jnp.float32),
            pltpu.VMEM((L,), jnp.int32),
            pltpu.SemaphoreType.DMA,
            pltpu.SemaphoreType.DMA,
        ],
    )


def _sel_body(shift, next_shift, hist_ref, t_ref, r_ref, *outs):
    h = hist_ref[...].astype(jnp.int32)
    hsum = jnp.sum(h, axis=0, keepdims=True)
    t = jnp.zeros((1, NBINS), jnp.int32)
    for l in range(L):
        t = t + hsum[:, l * NBINS:(l + 1) * NBINS]
    idx = lax.broadcasted_iota(jnp.int32, (1, NBINS), 1)
    csum = t
    s = 1
    while s < NBINS:
        csum = csum + jnp.where(idx >= s, pltpu.roll(csum, s, 1), 0)
        s *= 2
    suf = jnp.sum(t) - csum + t
    r = r_ref[0, 0]
    bstar = jnp.max(jnp.where(suf >= r, idx, -1))
    cnt_at = jnp.sum(jnp.where(idx == bstar, t, 0))
    suf_at = jnp.sum(jnp.where(idx == bstar, suf, 0))
    t_out = t_ref[0, 0] + (bstar << shift)
    r_out = r - (suf_at - cnt_at)
    if next_shift is None:
        tau_ref, = outs
        tau_ref[0, 0] = lax.bitcast_convert_type(t_out, jnp.float32)
    else:
        to_ref, ro_ref, lo_ref = outs
        to_ref[0, 0] = t_out
        ro_ref[0, 0] = r_out
        lo_ref[...] = jnp.full((1, L), t_out >> next_shift, jnp.int32)


def _select(shift, next_shift, hist, t_in, r_in, interpret=False):
    smem11 = pl.BlockSpec((1, 1), lambda: (0, 0), memory_space=pltpu.SMEM)
    if next_shift is None:
        out_specs = [smem11]
        out_shape = [jax.ShapeDtypeStruct((1, 1), jnp.float32)]
    else:
        out_specs = [smem11, smem11,
                     pl.BlockSpec((1, L), lambda: (0, 0))]
        out_shape = [jax.ShapeDtypeStruct((1, 1), jnp.int32),
                     jax.ShapeDtypeStruct((1, 1), jnp.int32),
                     jax.ShapeDtypeStruct((1, L), jnp.int32)]
    res = pl.pallas_call(
        functools.partial(_sel_body, shift, next_shift),
        in_specs=[
            pl.BlockSpec((NW, HIST_W), lambda: (0, 0)),
            smem11,
            smem11,
        ],
        out_specs=out_specs,
        out_shape=out_shape,
        interpret=interpret,
    )(hist, t_in, r_in)
    return res[0] if next_shift is None else res


def _dec_body(acts_ref, w_ref, bdec_ref, tau_ref, atk_ref, recon_ref):
    j = pl.program_id(0)
    tau = tau_ref[0, 0]
    a = acts_ref[...]
    atk = jnp.where(a >= tau, a, 0.0)
    atk_ref[...] = atk

    @pl.when(j == 0)
    def _():
        recon_ref[...] = jnp.broadcast_to(bdec_ref[...], (N, A))

    recon_ref[...] += jnp.dot(atk, w_ref[...],
                              preferred_element_type=jnp.float32)


def _decode(acts, w_dec, b_dec, tau, interpret=False):
    blk = 256
    return pl.pallas_call(
        _dec_body,
        grid=(D // blk,),
        in_specs=[
            pl.BlockSpec((N, blk), lambda j: (0, j)),
            pl.BlockSpec((blk, A), lambda j: (j, 0)),
            pl.BlockSpec((1, A), lambda j: (0, 0)),
            pl.BlockSpec((1, 1), lambda j: (0, 0), memory_space=pltpu.SMEM),
        ],
        out_specs=[
            pl.BlockSpec((N, blk), lambda j: (0, j)),
            pl.BlockSpec((N, A), lambda j: (0, 0)),
        ],
        out_shape=[
            jax.ShapeDtypeStruct((N, D), jnp.float32),
            jax.ShapeDtypeStruct((N, A), jnp.float32),
        ],
        interpret=interpret,
    )(acts, w_dec, b_dec.reshape(1, A), tau)


def _fin_body(recon_ref, xn_ref, mean_ref, std_ref, s1_ref, s0_ref,
              sae_ref, loss_ref, l2_ref, l1l_ref, l0n_ref, l1n_ref, fvu_ref,
              se_acc, xs_acc, x2_acc):
    i = pl.program_id(0)
    nblk = pl.num_programs(0)
    recon = recon_ref[...]
    xn = xn_ref[...]
    sae_ref[...] = recon * std_ref[...] + mean_ref[...]
    d = recon - xn

    @pl.when(i == 0)
    def _():
        se_acc[0, 0] = 0.0
        xs_acc[0, 0] = 0.0
        x2_acc[0, 0] = 0.0

    se_acc[0, 0] += jnp.sum(d * d)
    xs_acc[0, 0] += jnp.sum(xn)
    x2_acc[0, 0] += jnp.sum(xn * xn)

    @pl.when(i == nblk - 1)
    def _():
        m = N * A
        l2 = se_acc[0, 0] / m
        mean_xn = xs_acc[0, 0] / m
        x_var = (x2_acc[0, 0] - xs_acc[0, 0] * mean_xn) / (m - 1)
        l1n = s1_ref[0, 0] / N
        l0n = s0_ref[0, 0] / N
        l1l = L1_COEFF * l1n
        l2_ref[0, 0] = l2
        l1l_ref[0, 0] = l1l
        l0n_ref[0, 0] = l0n
        l1n_ref[0, 0] = l1n
        fvu_ref[0, 0] = l2 / (x_var + 1e-10)
        loss_ref[0, 0] = l2 + l1l


def _finalize(recon, xn, mean, std, s1, s0, interpret=False):
    blk = 512
    smem11 = pl.BlockSpec((1, 1), lambda i: (0, 0), memory_space=pltpu.SMEM)
    sca = jax.ShapeDtypeStruct((1, 1), jnp.float32)
    return pl.pallas_call(
        _fin_body,
        grid=(N // blk,),
        in_specs=[
            pl.BlockSpec((blk, A), lambda i: (i, 0)),
            pl.BlockSpec((blk, A), lambda i: (i, 0)),
            pl.BlockSpec((blk, 1), lambda i: (i, 0)),
            pl.BlockSpec((blk, 1), lambda i: (i, 0)),
            smem11,
            smem11,
        ],
        out_specs=[pl.BlockSpec((blk, A), lambda i: (i, 0))] + [smem11] * 6,
        out_shape=[jax.ShapeDtypeStruct((N, A), jnp.float32)] + [sca] * 6,
        scratch_shapes=[pltpu.SMEM((1, 1), jnp.float32)] * 3,
        interpret=interpret,
    )(recon, xn, mean, std, s1, s0)


def _threshold(acts):
    t = jnp.zeros((1, 1), jnp.int32)
    r = jnp.full((1, 1), K_TOT, jnp.int32)
    lo = jnp.zeros((16,), jnp.int32)
    for p in range(3):
        hist = _sc_hist(SHIFTS[p], SPANS[p])(acts, lo)
        if p < 2:
            t, r, lo2 = _select(SHIFTS[p], SHIFTS[p + 1], hist, t, r)
            lo = lo2.reshape(L)
        else:
            tau = _select(SHIFTS[p], None, hist, t, r)
    return tau


def kernel(x, W_enc, W_dec, b_enc, b_dec):
    xn, xe, mean, std = _prep(x, b_dec)
    acts, s1, s0 = _encode(xe, W_enc)
    tau = _threshold(acts)
    acts_topk, recon = _decode(acts, W_dec, b_dec, tau)
    sae_out, loss, l2, l1l, l0n, l1n, fvu = _finalize(
        recon, xn, mean, std, s1, s0)
    sc = lambda a: a.reshape(())
    return (sae_out, acts_topk, sc(loss), sc(l2), sc(l1l),
            sc(l0n), sc(l1n), sc(fvu))


# speedup vs baseline: 1.2886x; 1.2886x over previous
import functools

import jax
import jax.numpy as jnp
from jax import lax
from jax.experimental import pallas as pl
from jax.experimental.pallas import tpu as pltpu
from jax.experimental.pallas import tpu_sc as plsc

N = 4096
A = 768
D = 16384
TOP_K = 32
K_TOT = TOP_K * N
L1_COEFF = 0.0003

NC = 2
NS = 16
L = 16
NW = NC * NS

NBINS = 2048
HIST_W = L * NBINS
SHIFTS = (20, 10, 0)
SPANS = (2048, 1024, 1024)

ELEMS = N * D
PER_W = ELEMS // NW
WIN = 16384
N_WIN = PER_W // WIN


def _prep_body(x_ref, bdec_ref, xn_ref, xe_ref, mean_ref, std_ref):
    x = x_ref[...]
    mu = jnp.mean(x, axis=1, keepdims=True)
    xc = x - mu
    var = jnp.sum(xc * xc, axis=1, keepdims=True) / (A - 1)
    sd = jnp.sqrt(var)
    xn = xc / (sd + 1e-5)
    xn_ref[...] = xn
    xe_ref[...] = xn - bdec_ref[...]
    mean_ref[...] = mu
    std_ref[...] = sd


def _prep(x, b_dec, interpret=False):
    blk = 512
    return pl.pallas_call(
        _prep_body,
        grid=(N // blk,),
        in_specs=[
            pl.BlockSpec((blk, A), lambda i: (i, 0)),
            pl.BlockSpec((1, A), lambda i: (0, 0)),
        ],
        out_specs=[
            pl.BlockSpec((blk, A), lambda i: (i, 0)),
            pl.BlockSpec((blk, A), lambda i: (i, 0)),
            pl.BlockSpec((blk, 1), lambda i: (i, 0)),
            pl.BlockSpec((blk, 1), lambda i: (i, 0)),
        ],
        out_shape=[
            jax.ShapeDtypeStruct((N, A), jnp.float32),
            jax.ShapeDtypeStruct((N, A), jnp.float32),
            jax.ShapeDtypeStruct((N, 1), jnp.float32),
            jax.ShapeDtypeStruct((N, 1), jnp.float32),
        ],
        interpret=interpret,
    )(x, b_dec.reshape(1, A))


def _enc_body(xe_ref, w_ref, acts_ref, s1_ref, s0_ref):
    j = pl.program_id(0)
    a = jnp.dot(xe_ref[...], w_ref[...], preferred_element_type=jnp.float32)
    a = jnp.maximum(a, 0.0)
    acts_ref[...] = a

    @pl.when(j == 0)
    def _():
        s1_ref[0, 0] = 0.0
        s0_ref[0, 0] = 0.0

    s1_ref[0, 0] += jnp.sum(a)
    s0_ref[0, 0] += jnp.sum((a > 0.0).astype(jnp.float32))


def _encode(xe, w_enc, interpret=False):
    blk = 512
    return pl.pallas_call(
        _enc_body,
        grid=(D // blk,),
        in_specs=[
            pl.BlockSpec((N, A), lambda j: (0, 0)),
            pl.BlockSpec((A, blk), lambda j: (0, j)),
        ],
        out_specs=[
            pl.BlockSpec((N, blk), lambda j: (0, j)),
            pl.BlockSpec((1, 1), lambda j: (0, 0), memory_space=pltpu.SMEM),
            pl.BlockSpec((1, 1), lambda j: (0, 0), memory_space=pltpu.SMEM),
        ],
        out_shape=[
            jax.ShapeDtypeStruct((N, D), jnp.float32),
            jax.ShapeDtypeStruct((1, 1), jnp.float32),
            jax.ShapeDtypeStruct((1, 1), jnp.float32),
        ],
        interpret=interpret,
    )(xe, w_enc)


def _hist_body(shift, span, acts_hbm, lo_hbm, out_hbm,
               hist, buf0, buf1, lo_v, sem0, sem1):
    wid = lax.axis_index("s") * NC + lax.axis_index("c")
    base = wid * PER_W

    def zbody(i, _):
        hist[pl.ds(i * L, L)] = jnp.zeros((L,), jnp.int32)
        return 0
    lax.fori_loop(0, HIST_W // L, zbody, 0, unroll=4)

    pltpu.sync_copy(lo_hbm, lo_v)
    lo = lo_v[...]
    lane_base = lax.iota(jnp.int32, L) * NBINS
    ones = jnp.ones((L,), jnp.int32)

    def win_slice(g):
        return acts_hbm.at[pl.ds(base + g * WIN, WIN)]

    def compute(buf):
        @plsc.parallel_loop(0, WIN, step=L * 8)
        def _(c0):
            for u in range(8):
                    c = c0 + u * L
                    vals = buf[pl.ds(c, L)]
                    bits = plsc.bitcast(vals, jnp.int32)
                    if shift == SHIFTS[0]:
                        m = bits != 0
                        plsc.addupdate_scatter(
                            hist, [(bits >> shift) + lane_base], ones, mask=m)
                    else:
                        bin_ = (bits >> shift) - lo
                        m = bin_.astype(jnp.uint32) < jnp.uint32(span)
                        plsc.addupdate_scatter(
                            hist, [bin_ + lane_base], ones, mask=m)

    pltpu.make_async_copy(win_slice(0), buf0, sem0).start()
    pltpu.make_async_copy(win_slice(1), buf1, sem1).start()

    def gbody(gg, _):
        g0 = 2 * gg
        pltpu.make_async_copy(win_slice(g0), buf0, sem0).wait()
        compute(buf0)
        pltpu.make_async_copy(win_slice(g0 + 2), buf0, sem0).start()
        pltpu.make_async_copy(win_slice(g0 + 1), buf1, sem1).wait()
        compute(buf1)
        pltpu.make_async_copy(win_slice(g0 + 3), buf1, sem1).start()
        return 0

    lax.fori_loop(0, N_WIN // 2 - 1, gbody, 0)
    pltpu.make_async_copy(win_slice(N_WIN - 2), buf0, sem0).wait()
    compute(buf0)
    pltpu.make_async_copy(win_slice(N_WIN - 1), buf1, sem1).wait()
    compute(buf1)
    pltpu.sync_copy(hist, out_hbm.at[wid])


def _sc_hist(shift, span):
    mesh = plsc.VectorSubcoreMesh(core_axis_name="c", subcore_axis_name="s")
    return pl.kernel(
        functools.partial(_hist_body, shift, span),
        out_type=jax.ShapeDtypeStruct((NW, HIST_W), jnp.int32),
        mesh=mesh,
        compiler_params=pltpu.CompilerParams(needs_layout_passes=False),
        scratch_types=[
            pltpu.VMEM((HIST_W,), jnp.int32),
            pltpu.VMEM((WIN,), jnp.float32),
            pltpu.VMEM((WIN,), jnp.float32),
            pltpu.VMEM((L,), jnp.int32),
            pltpu.SemaphoreType.DMA,
            pltpu.SemaphoreType.DMA,
        ],
    )


def _sel_body(shift, next_shift, hist_ref, t_ref, r_ref, *outs):
    h = hist_ref[...].astype(jnp.int32)
    hsum = jnp.sum(h, axis=0, keepdims=True)
    t = jnp.zeros((1, NBINS), jnp.int32)
    for l in range(L):
        t = t + hsum[:, l * NBINS:(l + 1) * NBINS]
    idx = lax.broadcasted_iota(jnp.int32, (1, NBINS), 1)
    csum = t
    s = 1
    while s < NBINS:
        csum = csum + jnp.where(idx >= s, pltpu.roll(csum, s, 1), 0)
        s *= 2
    suf = jnp.sum(t) - csum + t
    r = r_ref[0, 0]
    bstar = jnp.max(jnp.where(suf >= r, idx, -1))
    cnt_at = jnp.sum(jnp.where(idx == bstar, t, 0))
    suf_at = jnp.sum(jnp.where(idx == bstar, suf, 0))
    t_out = t_ref[0, 0] + (bstar << shift)
    r_out = r - (suf_at - cnt_at)
    if next_shift is None:
        tau_ref, = outs
        tau_ref[0, 0] = lax.bitcast_convert_type(t_out, jnp.float32)
    else:
        to_ref, ro_ref, lo_ref = outs
        to_ref[0, 0] = t_out
        ro_ref[0, 0] = r_out
        lo_ref[...] = jnp.full((1, L), t_out >> next_shift, jnp.int32)


def _select(shift, next_shift, hist, t_in, r_in, interpret=False):
    smem11 = pl.BlockSpec((1, 1), lambda: (0, 0), memory_space=pltpu.SMEM)
    if next_shift is None:
        out_specs = [smem11]
        out_shape = [jax.ShapeDtypeStruct((1, 1), jnp.float32)]
    else:
        out_specs = [smem11, smem11,
                     pl.BlockSpec((1, L), lambda: (0, 0))]
        out_shape = [jax.ShapeDtypeStruct((1, 1), jnp.int32),
                     jax.ShapeDtypeStruct((1, 1), jnp.int32),
                     jax.ShapeDtypeStruct((1, L), jnp.int32)]
    res = pl.pallas_call(
        functools.partial(_sel_body, shift, next_shift),
        in_specs=[
            pl.BlockSpec((NW, HIST_W), lambda: (0, 0)),
            smem11,
            smem11,
        ],
        out_specs=out_specs,
        out_shape=out_shape,
        interpret=interpret,
    )(hist, t_in, r_in)
    return res[0] if next_shift is None else res


def _dec_body(acts_ref, w_ref, bdec_ref, tau_ref, atk_ref, recon_ref):
    j = pl.program_id(0)
    tau = tau_ref[0, 0]
    a = acts_ref[...]
    atk = jnp.where(a >= tau, a, 0.0)
    atk_ref[...] = atk

    @pl.when(j == 0)
    def _():
        recon_ref[...] = jnp.broadcast_to(bdec_ref[...], (N, A))

    recon_ref[...] += jnp.dot(atk, w_ref[...],
                              preferred_element_type=jnp.float32)


def _decode(acts, w_dec, b_dec, tau, interpret=False):
    blk = 256
    return pl.pallas_call(
        _dec_body,
        grid=(D // blk,),
        in_specs=[
            pl.BlockSpec((N, blk), lambda j: (0, j)),
            pl.BlockSpec((blk, A), lambda j: (j, 0)),
            pl.BlockSpec((1, A), lambda j: (0, 0)),
            pl.BlockSpec((1, 1), lambda j: (0, 0), memory_space=pltpu.SMEM),
        ],
        out_specs=[
            pl.BlockSpec((N, blk), lambda j: (0, j)),
            pl.BlockSpec((N, A), lambda j: (0, 0)),
        ],
        out_shape=[
            jax.ShapeDtypeStruct((N, D), jnp.float32),
            jax.ShapeDtypeStruct((N, A), jnp.float32),
        ],
        interpret=interpret,
    )(acts, w_dec, b_dec.reshape(1, A), tau)


def _fin_body(recon_ref, xn_ref, mean_ref, std_ref, s1_ref, s0_ref,
              sae_ref, loss_ref, l2_ref, l1l_ref, l0n_ref, l1n_ref, fvu_ref,
              se_acc, xs_acc, x2_acc):
    i = pl.program_id(0)
    nblk = pl.num_programs(0)
    recon = recon_ref[...]
    xn = xn_ref[...]
    sae_ref[...] = recon * std_ref[...] + mean_ref[...]
    d = recon - xn

    @pl.when(i == 0)
    def _():
        se_acc[0, 0] = 0.0
        xs_acc[0, 0] = 0.0
        x2_acc[0, 0] = 0.0

    se_acc[0, 0] += jnp.sum(d * d)
    xs_acc[0, 0] += jnp.sum(xn)
    x2_acc[0, 0] += jnp.sum(xn * xn)

    @pl.when(i == nblk - 1)
    def _():
        m = N * A
        l2 = se_acc[0, 0] / m
        mean_xn = xs_acc[0, 0] / m
        x_var = (x2_acc[0, 0] - xs_acc[0, 0] * mean_xn) / (m - 1)
        l1n = s1_ref[0, 0] / N
        l0n = s0_ref[0, 0] / N
        l1l = L1_COEFF * l1n
        l2_ref[0, 0] = l2
        l1l_ref[0, 0] = l1l
        l0n_ref[0, 0] = l0n
        l1n_ref[0, 0] = l1n
        fvu_ref[0, 0] = l2 / (x_var + 1e-10)
        loss_ref[0, 0] = l2 + l1l


def _finalize(recon, xn, mean, std, s1, s0, interpret=False):
    blk = 512
    smem11 = pl.BlockSpec((1, 1), lambda i: (0, 0), memory_space=pltpu.SMEM)
    sca = jax.ShapeDtypeStruct((1, 1), jnp.float32)
    return pl.pallas_call(
        _fin_body,
        grid=(N // blk,),
        in_specs=[
            pl.BlockSpec((blk, A), lambda i: (i, 0)),
            pl.BlockSpec((blk, A), lambda i: (i, 0)),
            pl.BlockSpec((blk, 1), lambda i: (i, 0)),
            pl.BlockSpec((blk, 1), lambda i: (i, 0)),
            smem11,
            smem11,
        ],
        out_specs=[pl.BlockSpec((blk, A), lambda i: (i, 0))] + [smem11] * 6,
        out_shape=[jax.ShapeDtypeStruct((N, A), jnp.float32)] + [sca] * 6,
        scratch_shapes=[pltpu.SMEM((1, 1), jnp.float32)] * 3,
        interpret=interpret,
    )(recon, xn, mean, std, s1, s0)


def _threshold(acts):
    acts_flat = acts.reshape(ELEMS)
    t = jnp.zeros((1, 1), jnp.int32)
    r = jnp.full((1, 1), K_TOT, jnp.int32)
    lo = jnp.zeros((16,), jnp.int32)
    for p in range(3):
        hist = _sc_hist(SHIFTS[p], SPANS[p])(acts_flat, lo)
        if p < 2:
            t, r, lo2 = _select(SHIFTS[p], SHIFTS[p + 1], hist, t, r)
            lo = lo2.reshape(L)
        else:
            tau = _select(SHIFTS[p], None, hist, t, r)
    return tau


def kernel(x, W_enc, W_dec, b_enc, b_dec):
    xn, xe, mean, std = _prep(x, b_dec)
    acts, s1, s0 = _encode(xe, W_enc)
    tau = _threshold(acts)
    acts_topk, recon = _decode(acts, W_dec, b_dec, tau)
    sae_out, loss, l2, l1l, l0n, l1n, fvu = _finalize(
        recon, xn, mean, std, s1, s0)
    sc = lambda a: a.reshape(())
    return (sae_out, acts_topk, sc(loss), sc(l2), sc(l1l),
            sc(l0n), sc(l1n), sc(fvu))
